# baseline (speedup 1.0000x reference)
import jax
import jax.numpy as jnp
from jax import lax
from jax.experimental import pallas as pl

_H = 64
_W = 64
_A = 9
_NT = _H * _W * _A // 128
_ROWS = _NT * 4


_GRID = 4
_BLK = _ROWS // _GRID


def _gen_body(o1_ref, o2_ref):
    i = pl.program_id(0)
    rp = lax.broadcasted_iota(jnp.int32, (_BLK, 128), 0) + i * _BLK
    l = lax.broadcasted_iota(jnp.int32, (_BLK, 128), 1)
    n = (rp >> 2) * 128 + l
    q = lax.shift_right_logical(n * 58255, 19)
    a = n - q * 9
    t = lax.shift_right_logical(a * 11, 5)
    s = a - t * 3
    cx = (q & 63).astype(jnp.float32) * 8.0 + 4.0
    cy = lax.shift_right_logical(q, 6).astype(jnp.float32) * 8.0 + 4.0
    e1 = s.astype(jnp.float32) * (1.0 / 3.0)
    e2 = t.astype(jnp.float32) * 0.5 - 0.5
    bw = 32.0 * jnp.exp2(e1 + e2)
    bh = 32.0 * jnp.exp2(e1 - e2)
    c_odd = (rp & 1) == 1
    c_low = (rp & 2) == 0
    u = jnp.where(c_odd, cy, cx)
    v = jnp.where(c_odd, bh, bw)
    o1_ref[...] = jnp.where(c_low, u, v)
    hv = v * jnp.where(c_low, -0.5, 0.5)
    o2_ref[...] = u + hv


def kernel(features):
    del features
    o1, o2 = pl.pallas_call(
        _gen_body,
        grid=(_GRID,),
        out_specs=(
            pl.BlockSpec((_BLK, 128), lambda i: (i, 0)),
            pl.BlockSpec((_BLK, 128), lambda i: (i, 0)),
        ),
        out_shape=(
            jax.ShapeDtypeStruct((_ROWS, 128), jnp.float32),
            jax.ShapeDtypeStruct((_ROWS, 128), jnp.float32),
        ),
    )()
    a1 = o1.reshape(_NT, 4, 128).transpose(0, 2, 1).reshape(_H * _W * _A, 4)
    a2 = o2.reshape(_NT, 4, 128).transpose(0, 2, 1).reshape(_H * _W * _A, 4)
    return a1, a2


# speedup vs baseline: 1.1897x; 1.1897x over previous
import jax
import jax.numpy as jnp
from jax import lax
from jax.experimental import pallas as pl

_H = 64
_W = 64
_A = 9
_NT = _H * _W * _A // 128
_ROWS = _NT * 4


_GRID = 2
_BLK = _ROWS // _GRID


def _gen_body(o1_ref, o2_ref):
    i = pl.program_id(0)
    rp = lax.broadcasted_iota(jnp.int32, (_BLK, 128), 0) + i * _BLK
    l = lax.broadcasted_iota(jnp.int32, (_BLK, 128), 1)
    n = (rp >> 2) * 128 + l
    q = lax.shift_right_logical(n * 58255, 19)
    a = n - q * 9
    t = lax.shift_right_logical(a * 11, 5)
    s = a - t * 3
    cx = (q & 63).astype(jnp.float32) * 8.0 + 4.0
    cy = lax.shift_right_logical(q, 6).astype(jnp.float32) * 8.0 + 4.0
    e1 = s.astype(jnp.float32) * (1.0 / 3.0)
    e2 = t.astype(jnp.float32) * 0.5 - 0.5
    bw = 32.0 * jnp.exp2(e1 + e2)
    bh = 32.0 * jnp.exp2(e1 - e2)
    c_odd = (rp & 1) == 1
    c_low = (rp & 2) == 0
    u = jnp.where(c_odd, cy, cx)
    v = jnp.where(c_odd, bh, bw)
    o1_ref[...] = jnp.where(c_low, u, v)
    hv = v * jnp.where(c_low, -0.5, 0.5)
    o2_ref[...] = u + hv


def kernel(features):
    del features
    o1, o2 = pl.pallas_call(
        _gen_body,
        grid=(_GRID,),
        out_specs=(
            pl.BlockSpec((_BLK, 128), lambda i: (i, 0)),
            pl.BlockSpec((_BLK, 128), lambda i: (i, 0)),
        ),
        out_shape=(
            jax.ShapeDtypeStruct((_ROWS, 128), jnp.float32),
            jax.ShapeDtypeStruct((_ROWS, 128), jnp.float32),
        ),
    )()
    a1 = o1.reshape(_NT, 4, 128).transpose(0, 2, 1).reshape(_H * _W * _A, 4)
    a2 = o2.reshape(_NT, 4, 128).transpose(0, 2, 1).reshape(_H * _W * _A, 4)
    return a1, a2
